# baseline (speedup 1.0000x reference)
import functools

import jax
import jax.numpy as jnp
from jax import lax
from jax.experimental import pallas as pl
from jax.experimental.pallas import tpu as pltpu
from jax.experimental.pallas import tpu_sc as plsc

USER_N = 100000
ITEM_N = 100000
DIM = 64
GENRES = 32
BATCH = 4096

_NC = 2
_NS = 16
_NW = _NC * _NS
_TCW = 128
_NTC = 9
_PW = _NTC * _TCW
_TC_FULL = USER_N // _TCW
_CLAMP = _TC_FULL * _TCW
_TAIL_N = USER_N - _CLAMP
_OUT_ROWS = BATCH + 8
_DUMP = BATCH


def _sc_body(user_h, item1_h, item2_h, xtu_h, xti_h,
             urows_h, i1rows_h, i2rows_h,
             idx_s, chunk_v, tail_v, vals_v, poss_v, pdma_v,
             rowbuf_v, wv_u, wp_u, wv_1, wp_1, wv_2, wp_2,
             sem0, sem1):
    sems = (sem0, sem1)
    wid = lax.axis_index("s") * _NC + lax.axis_index("c")
    w_lo = wid * 24 + jnp.minimum(wid, 13)
    w_sz = 24 + (wid < 13).astype(jnp.int32)
    w_hi = jnp.minimum(w_lo + w_sz, _TC_FULL)

    iota = lax.iota(jnp.int32, 16)
    w_base_r = w_lo * _TCW
    w_end_r = jnp.where(wid == _NW - 1, jnp.int32(USER_N), w_hi * _TCW)

    def big_scan(idx_ref, wvals, wposs):
        def chunk4(i, cnt):
            vs, bs, ms = [], [], []
            for s in range(4):
                v = idx_ref[pl.ds((i * 4 + s) * 16, 16)]
                vs.append(v)
                bs.append(iota + (i * 4 + s) * 16)
                ms.append((v >= w_base_r) & (v < w_end_r))
            mis = [m.astype(jnp.int32) for m in ms]
            csums = [plsc.cumsum(mi) for mi in mis]
            sums = [jnp.sum(mi) for mi in mis]
            base = cnt
            for s in range(4):
                pos = base + csums[s] - 1
                plsc.store_scatter(wvals, [pos], vs[s], mask=ms[s])
                plsc.store_scatter(wposs, [pos], bs[s], mask=ms[s])
                base = base + sums[s]
            return base

        return lax.fori_loop(0, BATCH // 64, chunk4, jnp.int32(0))

    pltpu.sync_copy(user_h, idx_s)
    cnt_u = big_scan(idx_s, wv_u, wp_u)
    pltpu.sync_copy(item1_h, idx_s)
    cnt_1 = big_scan(idx_s, wv_1, wp_1)
    pltpu.sync_copy(item2_h, idx_s)
    cnt_2 = big_scan(idx_s, wv_2, wp_2)

    def scan(pair, lo_r, hi_r, base_r):
        wvals, wposs, wcnt = pair

        def chunk(i, cnt):
            v = wvals[pl.ds(i * 16, 16)]
            b = wposs[pl.ds(i * 16, 16)]
            ok = (i * 16 + iota) < wcnt
            m = ok & (v >= lo_r) & (v < hi_r)
            mi = m.astype(jnp.int32)
            pos = cnt + plsc.cumsum(mi) - 1
            plsc.store_scatter(vals_v, [pos], v - base_r, mask=m)
            plsc.store_scatter(poss_v, [pos], b, mask=m)
            return cnt + jnp.sum(mi)

        return lax.fori_loop(0, (wcnt + 15) // 16, chunk, jnp.int32(0))

    def do_group(cnt, rows_h, src_ref, col_max, g, s, sync):
        posv = poss_v[pl.ds(g * 16, 16)]
        ok = (g * 16 + iota) < cnt
        posv = jnp.where(ok, posv, _DUMP)
        pdma_v[s, pl.ds(0, 16)] = posv
        rlv = vals_v[pl.ds(g * 16, 16)]
        for j in range(16):
            rl = lax.clamp(jnp.int32(0), rlv[j], jnp.int32(col_max))
            col = jnp.full((16,), rl, jnp.int32)
            for k in range(DIM // 16):
                gv = plsc.load_gather(src_ref, [iota + k * 16, col])
                rowbuf_v[s, j, pl.ds(k * 16, 16)] = gv
        if sync:
            pltpu.sync_copy(rowbuf_v.at[s], rows_h.at[pdma_v.at[s]])
        else:
            pltpu.async_copy(rowbuf_v.at[s], rows_h.at[pdma_v.at[s]],
                             sems[s])

    def extract(cnt, rows_h, src_ref, col_max):
        ngrp = (cnt + 15) // 16

        def block2(g0, carry):
            for s in range(2):
                g = g0 * 2 + s

                @pl.when(g < ngrp)
                def _one(g=g, s=s):
                    @pl.when(g0 > 0)
                    def _wait():
                        pltpu.make_async_copy(
                            rowbuf_v.at[s], rows_h.at[pdma_v.at[s]],
                            sems[s]).wait()

                    do_group(cnt, rows_h, src_ref, col_max, g, s, False)
            return carry

        lax.fori_loop(0, (ngrp + 1) // 2, block2, jnp.int32(0))
        for s in range(2):
            @pl.when(s < jnp.minimum(ngrp, 2))
            def _drain(s=s):
                pltpu.make_async_copy(
                    rowbuf_v.at[s], rows_h.at[pdma_v.at[s]], sems[s]).wait()

    def extract_sync(cnt, rows_h, src_ref, col_max):
        def group(g, carry):
            do_group(cnt, rows_h, src_ref, col_max, g, 0, True)
            return carry

        lax.fori_loop(0, (cnt + 15) // 16, group, jnp.int32(0))

    for p in range(3):
        pass_lo = w_lo + p * _NTC
        stream_tc = jnp.minimum(pass_lo, _TC_FULL - _NTC)
        pass_hi = jnp.minimum(pass_lo + _NTC, w_hi)

        @pl.when(pass_lo < w_hi)
        def _run(pass_lo=pass_lo, stream_tc=stream_tc, pass_hi=pass_hi):
            lo_r = pass_lo * _TCW
            hi_r = pass_hi * _TCW
            base_r = stream_tc * _TCW
            pltpu.sync_copy(xtu_h.at[:, pl.ds(base_r, _PW)], chunk_v)
            cnt = scan((wv_u, wp_u, cnt_u), lo_r, hi_r, base_r)
            extract(cnt, urows_h, chunk_v, _PW - 1)
            pltpu.sync_copy(xti_h.at[:, pl.ds(base_r, _PW)], chunk_v)
            cnt = scan((wv_1, wp_1, cnt_1), lo_r, hi_r, base_r)
            extract(cnt, i1rows_h, chunk_v, _PW - 1)
            cnt = scan((wv_2, wp_2, cnt_2), lo_r, hi_r, base_r)
            extract(cnt, i2rows_h, chunk_v, _PW - 1)

    @pl.when(wid == _NW - 1)
    def _tail():
        pltpu.sync_copy(xtu_h.at[:, pl.ds(_CLAMP, _TAIL_N)], tail_v)
        cnt = scan((wv_u, wp_u, cnt_u), _CLAMP, USER_N, _CLAMP)
        extract_sync(cnt, urows_h, tail_v, _TAIL_N - 1)
        pltpu.sync_copy(xti_h.at[:, pl.ds(_CLAMP, _TAIL_N)], tail_v)
        cnt = scan((wv_1, wp_1, cnt_1), _CLAMP, ITEM_N, _CLAMP)
        extract_sync(cnt, i1rows_h, tail_v, _TAIL_N - 1)
        cnt = scan((wv_2, wp_2, cnt_2), _CLAMP, ITEM_N, _CLAMP)
        extract_sync(cnt, i2rows_h, tail_v, _TAIL_N - 1)


def _sc_extract(user, item_1, item_2, xtu, xti):
    mesh = plsc.VectorSubcoreMesh(core_axis_name="c", subcore_axis_name="s")
    row_ty = jax.ShapeDtypeStruct((_OUT_ROWS, 128), jnp.float32)
    f = pl.kernel(
        _sc_body,
        out_type=(row_ty, row_ty, row_ty),
        mesh=mesh,
        compiler_params=pltpu.CompilerParams(
            needs_layout_passes=False,
            use_tc_tiling_on_sc=True,
        ),
        scratch_types=[
            pltpu.VMEM((BATCH,), jnp.int32),
            pltpu.VMEM((DIM, _PW), jnp.float32),
            pltpu.VMEM((DIM, _TAIL_N), jnp.float32),
            pltpu.VMEM((BATCH,), jnp.int32),
            pltpu.VMEM((BATCH,), jnp.int32),
            pltpu.VMEM((8, 16), jnp.int32),
            pltpu.VMEM((2, 16, 128), jnp.float32),
            pltpu.VMEM((BATCH,), jnp.int32),
            pltpu.VMEM((BATCH,), jnp.int32),
            pltpu.VMEM((BATCH,), jnp.int32),
            pltpu.VMEM((BATCH,), jnp.int32),
            pltpu.VMEM((BATCH,), jnp.int32),
            pltpu.VMEM((BATCH,), jnp.int32),
            pltpu.SemaphoreType.DMA,
            pltpu.SemaphoreType.DMA,
        ],
    )
    return f(user, item_1, item_2, xtu, xti)


def _head_body(u_ref, i1_ref, i2_ref, w_ref, b_ref, r_ref, g_ref):
    u = u_ref[...][:BATCH, :DIM]
    i1 = i1_ref[...][:BATCH, :DIM]
    i2 = i2_ref[...][:BATCH, :DIM]
    r_ref[...] = jnp.sum(u * i1, axis=1)
    g_ref[...] = (
        jnp.dot(i2, w_ref[...], preferred_element_type=jnp.float32,
                precision=jax.lax.Precision.HIGHEST)
        + b_ref[...]
    )


def _head_tc(urows, i1rows, i2rows, W_genre, b2):
    return pl.pallas_call(
        _head_body,
        out_shape=(
            jax.ShapeDtypeStruct((BATCH,), jnp.float32),
            jax.ShapeDtypeStruct((BATCH, GENRES), jnp.float32),
        ),
    )(urows, i1rows, i2rows, W_genre, b2)


def kernel(user, item_1, item_2, embd_user, embd_item, W_genre, b_genre):
    user = user.astype(jnp.int32)
    item_1 = item_1.astype(jnp.int32)
    item_2 = item_2.astype(jnp.int32)
    urows, i1rows, i2rows = _sc_extract(user, item_1, item_2,
                                        embd_user.T, embd_item.T)
    ratings, genres = _head_tc(urows, i1rows, i2rows, W_genre,
                               b_genre.reshape(1, GENRES))
    return (ratings, genres)


# speedup vs baseline: 1.0283x; 1.0283x over previous
import functools

import jax
import jax.numpy as jnp
from jax import lax
from jax.experimental import pallas as pl
from jax.experimental.pallas import tpu as pltpu
from jax.experimental.pallas import tpu_sc as plsc

USER_N = 100000
ITEM_N = 100000
DIM = 64
GENRES = 32
BATCH = 4096

_NC = 2
_NS = 16
_NW = _NC * _NS
_TCW = 128
_NTC = 9
_PW = _NTC * _TCW
_TC_FULL = USER_N // _TCW
_CLAMP = _TC_FULL * _TCW
_TAIL_N = USER_N - _CLAMP
_OUT_ROWS = BATCH + 8
_DUMP = BATCH


def _sc_body(user_h, item1_h, item2_h, xtu_h, xti_h,
             urows_h, i1rows_h, i2rows_h,
             idx_s, chunk_v, tail_v, vals_v, poss_v, pdma_v,
             rowbuf_v, wv_u, wp_u, wv_1, wp_1, wv_2, wp_2,
             sem0, sem1):
    sems = (sem0, sem1)
    wid = lax.axis_index("s") * _NC + lax.axis_index("c")
    w_lo = wid * 24 + jnp.minimum(wid, 13)
    w_sz = 24 + (wid < 13).astype(jnp.int32)
    w_hi = jnp.minimum(w_lo + w_sz, _TC_FULL)

    iota = lax.iota(jnp.int32, 16)
    w_base_r = w_lo * _TCW
    w_end_r = jnp.where(wid == _NW - 1, jnp.int32(USER_N), w_hi * _TCW)

    def big_scan(idx_ref, wvals, wposs):
        def chunk4(i, cnt):
            vs, bs, ms = [], [], []
            for s in range(4):
                v = idx_ref[pl.ds((i * 4 + s) * 16, 16)]
                vs.append(v)
                bs.append(iota + (i * 4 + s) * 16)
                ms.append((v >= w_base_r) & (v < w_end_r))
            mis = [m.astype(jnp.int32) for m in ms]
            csums = [plsc.cumsum(mi) for mi in mis]
            sums = [jnp.sum(mi) for mi in mis]
            base = cnt
            for s in range(4):
                pos = base + csums[s] - 1
                plsc.store_scatter(wvals, [pos], vs[s], mask=ms[s])
                plsc.store_scatter(wposs, [pos], bs[s], mask=ms[s])
                base = base + sums[s]
            return base

        return lax.fori_loop(0, BATCH // 64, chunk4, jnp.int32(0))

    pltpu.sync_copy(user_h, idx_s)
    cnt_u = big_scan(idx_s, wv_u, wp_u)
    pltpu.sync_copy(item1_h, idx_s)
    cnt_1 = big_scan(idx_s, wv_1, wp_1)
    pltpu.sync_copy(item2_h, idx_s)
    cnt_2 = big_scan(idx_s, wv_2, wp_2)

    def scan(pair, lo_r, hi_r, base_r):
        wvals, wposs, wcnt = pair

        def chunk(i, cnt):
            v = wvals[pl.ds(i * 16, 16)]
            b = wposs[pl.ds(i * 16, 16)]
            ok = (i * 16 + iota) < wcnt
            m = ok & (v >= lo_r) & (v < hi_r)
            mi = m.astype(jnp.int32)
            pos = cnt + plsc.cumsum(mi) - 1
            plsc.store_scatter(vals_v, [pos], v - base_r, mask=m)
            plsc.store_scatter(poss_v, [pos], b, mask=m)
            return cnt + jnp.sum(mi)

        return lax.fori_loop(0, (wcnt + 15) // 16, chunk, jnp.int32(0))

    def do_group(cnt, rows_h, src_ref, col_max, g, s, sync):
        posv = poss_v[pl.ds(g * 16, 16)]
        ok = (g * 16 + iota) < cnt
        posv = jnp.where(ok, posv, _DUMP)
        pdma_v[s, pl.ds(0, 16)] = posv
        rlv = vals_v[pl.ds(g * 16, 16)]
        for j in range(16):
            rl = lax.clamp(jnp.int32(0), rlv[j], jnp.int32(col_max))
            col = jnp.full((16,), rl, jnp.int32)
            for k in range(DIM // 16):
                gv = plsc.load_gather(src_ref, [iota + k * 16, col])
                rowbuf_v[s, j, pl.ds(k * 16, 16)] = gv
        if sync:
            pltpu.sync_copy(rowbuf_v.at[s], rows_h.at[pdma_v.at[s]])
        else:
            pltpu.async_copy(rowbuf_v.at[s], rows_h.at[pdma_v.at[s]],
                             sems[s])

    def extract(cnt, rows_h, src_ref, col_max):
        def group(g, carry):
            do_group(cnt, rows_h, src_ref, col_max, g, 0, True)
            return carry

        lax.fori_loop(0, (cnt + 15) // 16, group, jnp.int32(0))

    extract_sync = extract

    for p in range(3):
        pass_lo = w_lo + p * _NTC
        stream_tc = jnp.minimum(pass_lo, _TC_FULL - _NTC)
        pass_hi = jnp.minimum(pass_lo + _NTC, w_hi)

        @pl.when(pass_lo < w_hi)
        def _run(pass_lo=pass_lo, stream_tc=stream_tc, pass_hi=pass_hi):
            lo_r = pass_lo * _TCW
            hi_r = pass_hi * _TCW
            base_r = stream_tc * _TCW
            pltpu.sync_copy(xtu_h.at[:, pl.ds(base_r, _PW)], chunk_v)
            cnt = scan((wv_u, wp_u, cnt_u), lo_r, hi_r, base_r)
            extract(cnt, urows_h, chunk_v, _PW - 1)
            pltpu.sync_copy(xti_h.at[:, pl.ds(base_r, _PW)], chunk_v)
            cnt = scan((wv_1, wp_1, cnt_1), lo_r, hi_r, base_r)
            extract(cnt, i1rows_h, chunk_v, _PW - 1)
            cnt = scan((wv_2, wp_2, cnt_2), lo_r, hi_r, base_r)
            extract(cnt, i2rows_h, chunk_v, _PW - 1)

    @pl.when(wid == _NW - 1)
    def _tail():
        pltpu.sync_copy(xtu_h.at[:, pl.ds(_CLAMP, _TAIL_N)], tail_v)
        cnt = scan((wv_u, wp_u, cnt_u), _CLAMP, USER_N, _CLAMP)
        extract_sync(cnt, urows_h, tail_v, _TAIL_N - 1)
        pltpu.sync_copy(xti_h.at[:, pl.ds(_CLAMP, _TAIL_N)], tail_v)
        cnt = scan((wv_1, wp_1, cnt_1), _CLAMP, ITEM_N, _CLAMP)
        extract_sync(cnt, i1rows_h, tail_v, _TAIL_N - 1)
        cnt = scan((wv_2, wp_2, cnt_2), _CLAMP, ITEM_N, _CLAMP)
        extract_sync(cnt, i2rows_h, tail_v, _TAIL_N - 1)


def _sc_extract(user, item_1, item_2, xtu, xti):
    mesh = plsc.VectorSubcoreMesh(core_axis_name="c", subcore_axis_name="s")
    row_ty = jax.ShapeDtypeStruct((_OUT_ROWS, 128), jnp.float32)
    f = pl.kernel(
        _sc_body,
        out_type=(row_ty, row_ty, row_ty),
        mesh=mesh,
        compiler_params=pltpu.CompilerParams(
            needs_layout_passes=False,
            use_tc_tiling_on_sc=True,
        ),
        scratch_types=[
            pltpu.VMEM((BATCH,), jnp.int32),
            pltpu.VMEM((DIM, _PW), jnp.float32),
            pltpu.VMEM((DIM, _TAIL_N), jnp.float32),
            pltpu.VMEM((BATCH,), jnp.int32),
            pltpu.VMEM((BATCH,), jnp.int32),
            pltpu.VMEM((8, 16), jnp.int32),
            pltpu.VMEM((2, 16, 128), jnp.float32),
            pltpu.VMEM((BATCH,), jnp.int32),
            pltpu.VMEM((BATCH,), jnp.int32),
            pltpu.VMEM((BATCH,), jnp.int32),
            pltpu.VMEM((BATCH,), jnp.int32),
            pltpu.VMEM((BATCH,), jnp.int32),
            pltpu.VMEM((BATCH,), jnp.int32),
            pltpu.SemaphoreType.DMA,
            pltpu.SemaphoreType.DMA,
        ],
    )
    return f(user, item_1, item_2, xtu, xti)


def _head_body(u_ref, i1_ref, i2_ref, w_ref, b_ref, r_ref, g_ref):
    u = u_ref[...][:BATCH, :DIM]
    i1 = i1_ref[...][:BATCH, :DIM]
    i2 = i2_ref[...][:BATCH, :DIM]
    r_ref[...] = jnp.sum(u * i1, axis=1)
    g_ref[...] = (
        jnp.dot(i2, w_ref[...], preferred_element_type=jnp.float32,
                precision=jax.lax.Precision.HIGHEST)
        + b_ref[...]
    )


def _head_tc(urows, i1rows, i2rows, W_genre, b2):
    return pl.pallas_call(
        _head_body,
        out_shape=(
            jax.ShapeDtypeStruct((BATCH,), jnp.float32),
            jax.ShapeDtypeStruct((BATCH, GENRES), jnp.float32),
        ),
    )(urows, i1rows, i2rows, W_genre, b2)


def kernel(user, item_1, item_2, embd_user, embd_item, W_genre, b_genre):
    user = user.astype(jnp.int32)
    item_1 = item_1.astype(jnp.int32)
    item_2 = item_2.astype(jnp.int32)
    urows, i1rows, i2rows = _sc_extract(user, item_1, item_2,
                                        embd_user.T, embd_item.T)
    ratings, genres = _head_tc(urows, i1rows, i2rows, W_genre,
                               b_genre.reshape(1, GENRES))
    return (ratings, genres)
